# Initial kernel scaffold; baseline (speedup 1.0000x reference)
import jax
import jax.numpy as jnp
from jax.experimental import pallas as pl


def kernel(z, edge_index):
    raise NotImplementedError("write your pallas kernel here")


# speedup vs baseline: 1.0983x; 1.0983x over previous
import functools

import jax
import jax.numpy as jnp
from jax import lax
from jax.experimental import pallas as pl
from jax.experimental.pallas import tpu as pltpu
from jax.experimental.pallas import tpu_sc as plsc

_NC = 2
_NS = 16
_NW = _NC * _NS
_L = 16

_CH = 80


def _gae_decode(z, src_idx, dst_idx):
    n, d = z.shape
    e = src_idx.shape[0]
    epw = e // _NW
    nchunk = epw // _CH
    groups = _CH // _L

    mesh = plsc.VectorSubcoreMesh(core_axis_name="c", subcore_axis_name="s")

    @functools.partial(
        pl.kernel,
        mesh=mesh,
        compiler_params=pltpu.CompilerParams(needs_layout_passes=False),
        out_type=jax.ShapeDtypeStruct((e,), jnp.float32),
        scratch_types=[
            pltpu.VMEM((_CH,), jnp.int32),
            pltpu.VMEM((_CH,), jnp.int32),
            pltpu.VMEM((_CH, d), jnp.float32),
            pltpu.VMEM((_CH, d), jnp.float32),
            pltpu.VMEM((_CH,), jnp.float32),
            pltpu.SemaphoreType.DMA,
            pltpu.SemaphoreType.DMA,
        ],
    )
    def decode(z_hbm, sidx_hbm, didx_hbm, out_hbm,
               sidx_v, didx_v, srows_v, drows_v, outc_v, sem_s, sem_d):
        wid = lax.axis_index("s") * _NC + lax.axis_index("c")
        wbase = wid * epw

        def chunk_body(c, carry):
            base = wbase + c * _CH
            pltpu.sync_copy(sidx_hbm.at[pl.ds(base, _CH)], sidx_v)
            pltpu.sync_copy(didx_hbm.at[pl.ds(base, _CH)], didx_v)
            cp_s = pltpu.async_copy(z_hbm.at[sidx_v], srows_v, sem_s)
            cp_d = pltpu.async_copy(z_hbm.at[didx_v], drows_v, sem_d)
            cp_s.wait()
            cp_d.wait()
            for g in range(groups):
                e_vec = lax.iota(jnp.int32, _L) + g * _L

                def d_body(k, acc):
                    d_vec = jnp.full((_L,), k, jnp.int32)
                    sv = plsc.load_gather(srows_v, [e_vec, d_vec])
                    dv = plsc.load_gather(drows_v, [e_vec, d_vec])
                    return acc + sv * dv

                acc = lax.fori_loop(0, d, d_body,
                                    jnp.zeros((_L,), jnp.float32), unroll=8)
                outc_v[pl.ds(g * _L, _L)] = 1.0 / (1.0 + jnp.exp(-acc))
            pltpu.sync_copy(outc_v, out_hbm.at[pl.ds(base, _CH)])
            return carry

        lax.fori_loop(0, nchunk, chunk_body, 0)

    return decode(z, src_idx, dst_idx)


def kernel(z, edge_index):
    ei = edge_index.astype(jnp.int32)
    return _gae_decode(z.astype(jnp.float32), ei[0], ei[1])


# speedup vs baseline: 1.1005x; 1.0020x over previous
import functools

import jax
import jax.numpy as jnp
from jax import lax
from jax.experimental import pallas as pl
from jax.experimental.pallas import tpu as pltpu
from jax.experimental.pallas import tpu_sc as plsc

_NC = 2
_NS = 16
_NW = _NC * _NS
_L = 16

_CH = 80


def _gae_decode(z, src_idx, dst_idx):
    n, d = z.shape
    e = src_idx.shape[0]
    epw = e // _NW
    nchunk = epw // _CH
    groups = _CH // _L

    mesh = plsc.VectorSubcoreMesh(core_axis_name="c", subcore_axis_name="s")

    @functools.partial(
        pl.kernel,
        mesh=mesh,
        compiler_params=pltpu.CompilerParams(needs_layout_passes=False),
        out_type=jax.ShapeDtypeStruct((e,), jnp.float32),
        scratch_types=[
            pltpu.VMEM((_CH,), jnp.int32),
            pltpu.VMEM((_CH,), jnp.int32),
            pltpu.VMEM((_CH, d + 1), jnp.float32),
            pltpu.VMEM((_CH, d + 1), jnp.float32),
            pltpu.VMEM((_CH,), jnp.float32),
            pltpu.SemaphoreType.DMA,
            pltpu.SemaphoreType.DMA,
        ],
    )
    def decode(z_hbm, sidx_hbm, didx_hbm, out_hbm,
               sidx_v, didx_v, srows_v, drows_v, outc_v, sem_s, sem_d):
        wid = lax.axis_index("s") * _NC + lax.axis_index("c")
        wbase = wid * epw

        def chunk_body(c, carry):
            base = wbase + c * _CH
            pltpu.sync_copy(sidx_hbm.at[pl.ds(base, _CH)], sidx_v)
            pltpu.sync_copy(didx_hbm.at[pl.ds(base, _CH)], didx_v)
            cp_s = pltpu.async_copy(z_hbm.at[sidx_v],
                                    srows_v.at[:, pl.ds(0, d)], sem_s)
            cp_d = pltpu.async_copy(z_hbm.at[didx_v],
                                    drows_v.at[:, pl.ds(0, d)], sem_d)
            cp_s.wait()
            cp_d.wait()
            for g in range(groups):
                e_vec = lax.iota(jnp.int32, _L) + g * _L

                def d_body(k, acc):
                    d_vec = jnp.full((_L,), k, jnp.int32)
                    sv = plsc.load_gather(srows_v, [e_vec, d_vec])
                    dv = plsc.load_gather(drows_v, [e_vec, d_vec])
                    return acc + sv * dv

                acc = lax.fori_loop(0, d, d_body,
                                    jnp.zeros((_L,), jnp.float32), unroll=8)
                outc_v[pl.ds(g * _L, _L)] = 1.0 / (1.0 + jnp.exp(-acc))
            pltpu.sync_copy(outc_v, out_hbm.at[pl.ds(base, _CH)])
            return carry

        lax.fori_loop(0, nchunk, chunk_body, 0)

    return decode(z, src_idx, dst_idx)


def kernel(z, edge_index):
    ei = edge_index.astype(jnp.int32)
    return _gae_decode(z.astype(jnp.float32), ei[0], ei[1])
